# Initial kernel scaffold; baseline (speedup 1.0000x reference)
import jax
import jax.numpy as jnp
from jax.experimental import pallas as pl


def kernel(input, token_table):
    raise NotImplementedError("write your pallas kernel here")


# speedup vs baseline: 1.0565x; 1.0565x over previous
import functools
import math

import jax
import jax.numpy as jnp
import numpy as np
from jax import lax
from jax.experimental import pallas as pl
from jax.experimental.pallas import tpu as pltpu
from jax.experimental.pallas import tpu_sc as plsc

VOCAB = 1000000
SEQ_LEN = 2048
D_EMBED = 128
BATCH = 4
SCALE = math.sqrt(float(D_EMBED))

NUM_CORES = 2
NUM_SUBCORES = 16
NW = NUM_CORES * NUM_SUBCORES
B_TOTAL = BATCH * SEQ_LEN
B_PER_W = B_TOTAL // NW
G_CHUNK = 128
N_CHUNKS = B_PER_W // G_CHUNK
LANES = 16


def _positional_table() -> np.ndarray:
    pos = np.arange(SEQ_LEN)[:, None].astype(np.float32)
    i = np.arange(D_EMBED)[None, :].astype(np.float32)
    angle_rates = 1.0 / np.power(
        10000.0, (2.0 * np.floor(i / 2.0)) / float(D_EMBED))
    angles = pos * angle_rates
    pe = np.zeros((SEQ_LEN, D_EMBED), dtype=np.float32)
    pe[:, 0::2] = np.sin(angles[:, 0::2])
    pe[:, 1::2] = np.cos(angles[:, 1::2])
    return pe


_PE_NP = _positional_table()


def _make_sc_kernel():
    mesh = plsc.VectorSubcoreMesh(
        core_axis_name="c", subcore_axis_name="s")

    @functools.partial(
        pl.kernel,
        mesh=mesh,
        out_type=jax.ShapeDtypeStruct((B_TOTAL, D_EMBED), jnp.float32),
        scratch_types=[
            pltpu.VMEM((N_CHUNKS, G_CHUNK), jnp.int32),
            pltpu.VMEM((B_PER_W, D_EMBED), jnp.float32),
            pltpu.VMEM((B_PER_W, D_EMBED), jnp.float32),
            pltpu.SemaphoreType.DMA,
        ],
    )
    def emb_kernel(table_hbm, idx_hbm, pe_hbm, out_hbm,
                   idx_v, rows_v, pe_v, sem):
        wid = lax.axis_index("s") * NUM_CORES + lax.axis_index("c")
        base = wid * B_PER_W
        pltpu.sync_copy(idx_hbm.at[wid], idx_v)
        copies = [
            pltpu.async_copy(
                table_hbm.at[idx_v.at[j]],
                rows_v.at[pl.ds(j * G_CHUNK, G_CHUNK)],
                sem,
            )
            for j in range(N_CHUNKS)
        ]
        pos0 = lax.rem(base, SEQ_LEN)
        pltpu.sync_copy(pe_hbm.at[pl.ds(pos0, B_PER_W)], pe_v)
        for cp in copies:
            cp.wait()

        def row_body(r, carry):
            for j in range(D_EMBED // LANES):
                sl = pl.ds(j * LANES, LANES)
                rows_v[r, sl] = rows_v[r, sl] * SCALE + pe_v[r, sl]
            return carry

        lax.fori_loop(0, B_PER_W, row_body, 0)
        pltpu.sync_copy(rows_v, out_hbm.at[pl.ds(base, B_PER_W)])

    return emb_kernel


_EMB_KERNEL = _make_sc_kernel()


def kernel(input, token_table):
    idx = input.reshape(NW, N_CHUNKS, G_CHUNK).astype(jnp.int32)
    pe = jnp.asarray(_PE_NP)
    out = _EMB_KERNEL(token_table, idx, pe)
    return out.reshape(BATCH, SEQ_LEN, D_EMBED)


# speedup vs baseline: 1.1212x; 1.0612x over previous
import functools
import math

import jax
import jax.numpy as jnp
import numpy as np
from jax import lax
from jax.experimental import pallas as pl
from jax.experimental.pallas import tpu as pltpu
from jax.experimental.pallas import tpu_sc as plsc

VOCAB = 1000000
SEQ_LEN = 2048
D_EMBED = 128
BATCH = 4
SCALE = math.sqrt(float(D_EMBED))

NUM_CORES = 2
NUM_SUBCORES = 16
NW = NUM_CORES * NUM_SUBCORES
B_TOTAL = BATCH * SEQ_LEN
B_PER_W = B_TOTAL // NW
W_PER_BATCH = SEQ_LEN // B_PER_W
N_CHUNKS = 4
C_ROWS = B_PER_W // N_CHUNKS
LANES = 16


def _positional_table() -> np.ndarray:
    pos = np.arange(SEQ_LEN)[:, None].astype(np.float32)
    i = np.arange(D_EMBED)[None, :].astype(np.float32)
    angle_rates = 1.0 / np.power(
        10000.0, (2.0 * np.floor(i / 2.0)) / float(D_EMBED))
    angles = pos * angle_rates
    pe = np.zeros((SEQ_LEN, D_EMBED), dtype=np.float32)
    pe[:, 0::2] = np.sin(angles[:, 0::2])
    pe[:, 1::2] = np.cos(angles[:, 1::2])
    return pe


_PE_NP = _positional_table()


def _make_sc_kernel():
    mesh = plsc.VectorSubcoreMesh(
        core_axis_name="c", subcore_axis_name="s")

    @functools.partial(
        pl.kernel,
        mesh=mesh,
        out_type=jax.ShapeDtypeStruct((BATCH, SEQ_LEN, D_EMBED), jnp.float32),
        scratch_types=[
            pltpu.VMEM((B_PER_W,), jnp.int32),
            pltpu.VMEM((N_CHUNKS, C_ROWS, D_EMBED), jnp.float32),
            pltpu.VMEM((B_PER_W, D_EMBED), jnp.float32),
            pltpu.SemaphoreType.DMA,
            pltpu.SemaphoreType.DMA,
            pltpu.SemaphoreType.DMA,
            pltpu.SemaphoreType.DMA,
            pltpu.SemaphoreType.DMA,
            pltpu.SemaphoreType.DMA,
        ],
    )
    def emb_kernel(table_hbm, idx_hbm, pe_hbm, out_hbm,
                   idx_v, rows_v, pe_v, g0, g1, g2, g3, pe_sem, w_sem):
        wid = lax.axis_index("s") * NUM_CORES + lax.axis_index("c")
        b = wid // W_PER_BATCH
        col0 = lax.rem(wid, W_PER_BATCH) * B_PER_W
        pltpu.sync_copy(idx_hbm.at[b, pl.ds(col0, B_PER_W)], idx_v)
        gsems = [g0, g1, g2, g3]
        gathers = [
            pltpu.async_copy(
                table_hbm.at[idx_v.at[pl.ds(c * C_ROWS, C_ROWS)]],
                rows_v.at[c],
                gsems[c],
            )
            for c in range(N_CHUNKS)
        ]
        pe_cp = pltpu.async_copy(pe_hbm.at[pl.ds(col0, B_PER_W)], pe_v, pe_sem)
        pe_cp.wait()
        writes = []
        for c in range(N_CHUNKS):
            gathers[c].wait()

            def row_body(r, carry, c=c):
                for j in range(D_EMBED // LANES):
                    sl = pl.ds(j * LANES, LANES)
                    rows_v[c, r, sl] = (rows_v[c, r, sl] * SCALE
                                        + pe_v[c * C_ROWS + r, sl])
                return carry

            lax.fori_loop(0, C_ROWS, row_body, 0)
            writes.append(pltpu.async_copy(
                rows_v.at[c],
                out_hbm.at[b, pl.ds(col0 + c * C_ROWS, C_ROWS)],
                w_sem,
            ))
        for w in writes:
            w.wait()

    return emb_kernel


_EMB_KERNEL = _make_sc_kernel()


def kernel(input, token_table):
    pe = jnp.asarray(_PE_NP)
    return _EMB_KERNEL(token_table, input, pe)
